# Initial kernel scaffold; baseline (speedup 1.0000x reference)
import jax
import jax.numpy as jnp
from jax.experimental import pallas as pl


def kernel(x, stem_w, stem_gamma, stem_beta, l1b0_c1_w, l1b0_c1_g, l1b0_c1_b, l1b0_c2_w, l1b0_c2_g, l1b0_c2_b, l1b1_c1_w, l1b1_c1_g, l1b1_c1_b, l1b1_c2_w, l1b1_c2_g, l1b1_c2_b, l2b0_c1_w, l2b0_c1_g, l2b0_c1_b, l2b0_c2_w, l2b0_c2_g, l2b0_c2_b, l2b0_ds_w, l2b0_ds_g, l2b0_ds_b, l2b1_c1_w, l2b1_c1_g, l2b1_c1_b, l2b1_c2_w, l2b1_c2_g, l2b1_c2_b, l3b0_c1_w, l3b0_c1_g, l3b0_c1_b, l3b0_c2_w, l3b0_c2_g, l3b0_c2_b, l3b0_ds_w, l3b0_ds_g, l3b0_ds_b, l3b1_c1_w, l3b1_c1_g, l3b1_c1_b, l3b1_c2_w, l3b1_c2_g, l3b1_c2_b, l4b0_c1_w, l4b0_c1_g, l4b0_c1_b, l4b0_c2_w, l4b0_c2_g, l4b0_c2_b, l4b0_ds_w, l4b0_ds_g, l4b0_ds_b, l4b1_c1_w, l4b1_c1_g, l4b1_c1_b, l4b1_c2_w, l4b1_c2_g, l4b1_c2_b, fc_w, fc_g, fc_b, cls0_w, cls0_b, cls1_w, cls1_b, cls2_w, cls2_b, con0_w, con0_b, con1_w, con1_b, ft_w, ft_b):
    raise NotImplementedError("write your pallas kernel here")


# speedup vs baseline: 1.0184x; 1.0184x over previous
import functools

import jax
import jax.numpy as jnp
from jax.experimental import pallas as pl
from jax.experimental.pallas import tpu as pltpu

_EPS = 1e-5


def _rup(v, m):
    return (v + m - 1) // m * m


def _fused_mm_bn_body(*refs, relu, has_res, inv_m, tm):
    if has_res:
        a_ref, w_ref, g_ref, b_ref, r_ref, o_ref, y_scr, s_scr, ss_scr = refs
    else:
        a_ref, w_ref, g_ref, b_ref, o_ref, y_scr, s_scr, ss_scr = refs
        r_ref = None
    ph = pl.program_id(0)
    it = pl.program_id(1)

    @pl.when(ph == 0)
    def _matmul_phase():
        acc = jnp.dot(a_ref[...], w_ref[...], preferred_element_type=jnp.float32)
        y_scr[pl.ds(it * tm, tm), :] = acc
        cs = jnp.sum(acc, axis=0, keepdims=True)
        css = jnp.sum(acc * acc, axis=0, keepdims=True)

        @pl.when(it == 0)
        def _init():
            s_scr[...] = cs
            ss_scr[...] = css

        @pl.when(it != 0)
        def _accum():
            s_scr[...] = s_scr[...] + cs
            ss_scr[...] = ss_scr[...] + css

    @pl.when(ph == 1)
    def _normalize_phase():
        mean = s_scr[...] * inv_m
        var = jnp.maximum(ss_scr[...] * inv_m - mean * mean, 0.0)
        scale = g_ref[...] * jax.lax.rsqrt(var + _EPS)
        shift = b_ref[...] - mean * scale
        y = y_scr[pl.ds(it * tm, tm), :] * scale + shift
        if has_res:
            y = y + r_ref[...].astype(jnp.float32)
        if relu:
            y = jnp.maximum(y, 0.0)
        o_ref[...] = y.astype(o_ref.dtype)


def _mm_bn(a, w, gamma, beta, residual=None, relu=True):
    M, K = a.shape
    Nc = w.shape[1]
    Kp, Np = _rup(K, 128), _rup(Nc, 128)

    tm = min(_rup(M, 16), 2048)
    while tm > 256 and (4 * tm * Kp + _rup(M, tm) * Np * 4
                        + 2 * Kp * Np) > 20 * 1024 * 1024:
        tm //= 2
    Mp = _rup(M, tm)
    nt = Mp // tm

    a_p = jnp.pad(a.astype(jnp.bfloat16), ((0, Mp - M), (0, Kp - K)))
    w_p = jnp.pad(w.astype(jnp.bfloat16), ((0, Kp - K), (0, Np - Nc)))
    g_p = jnp.pad(gamma.astype(jnp.float32), (0, Np - Nc)).reshape(1, Np)
    b_p = jnp.pad(beta.astype(jnp.float32), (0, Np - Nc)).reshape(1, Np)

    args = [a_p, w_p, g_p, b_p]
    in_specs = [
        pl.BlockSpec((tm, Kp), lambda p, i: (i * (1 - p), 0)),
        pl.BlockSpec((Kp, Np), lambda p, i: (0, 0)),
        pl.BlockSpec((1, Np), lambda p, i: (0, 0)),
        pl.BlockSpec((1, Np), lambda p, i: (0, 0)),
    ]
    if residual is not None:
        r_p = jnp.pad(residual.astype(jnp.bfloat16),
                      ((0, Mp - M), (0, Np - Nc)))
        args.append(r_p)
        in_specs.append(pl.BlockSpec((tm, Np), lambda p, i: (i * p, 0)))

    out = pl.pallas_call(
        functools.partial(_fused_mm_bn_body, relu=relu,
                          has_res=residual is not None,
                          inv_m=1.0 / float(M), tm=tm),
        out_shape=jax.ShapeDtypeStruct((Mp, Np), jnp.bfloat16),
        grid=(2, nt),
        in_specs=in_specs,
        out_specs=pl.BlockSpec((tm, Np), lambda p, i: (i * p, 0)),
        scratch_shapes=[pltpu.VMEM((Mp, Np), jnp.float32),
                        pltpu.VMEM((1, Np), jnp.float32),
                        pltpu.VMEM((1, Np), jnp.float32)],
        compiler_params=pltpu.CompilerParams(
            dimension_semantics=("arbitrary", "arbitrary")),
    )(*args)
    return out[:M, :Nc]


def _patches(x, ksize, stride, padding):
    N, D, H, W, C = x.shape
    kd, kh, kw = ksize
    sd, sh, sw = stride
    pd, ph, pw = padding
    Do = (D + 2 * pd - kd) // sd + 1
    Ho = (H + 2 * ph - kh) // sh + 1
    Wo = (W + 2 * pw - kw) // sw + 1
    if ksize == (1, 1, 1):
        sl = x[:, ::sd, ::sh, ::sw, :]
        return sl.reshape(N * Do * Ho * Wo, C), (Do, Ho, Wo)
    xp = jnp.pad(x, ((0, 0), (pd, pd), (ph, ph), (pw, pw), (0, 0)))
    cols = []
    for i in range(kd):
        for j in range(kh):
            for q in range(kw):
                cols.append(xp[:, i:i + sd * (Do - 1) + 1:sd,
                               j:j + sh * (Ho - 1) + 1:sh,
                               q:q + sw * (Wo - 1) + 1:sw, :])
    A = jnp.concatenate(cols, axis=-1)
    return A.reshape(N * Do * Ho * Wo, kd * kh * kw * C), (Do, Ho, Wo)


def _conv(x, w, g, b, ksize, stride, padding, relu=True, residual=None):
    A, (Do, Ho, Wo) = _patches(x, ksize, stride, padding)
    N = x.shape[0]
    Nc = w.shape[1]
    res = residual.reshape(-1, Nc) if residual is not None else None
    y = _mm_bn(A, w, g, b, residual=res, relu=relu)
    return y.reshape(N, Do, Ho, Wo, Nc)


def _basic_block(h, c1, c2, ds, stride):
    s3 = (stride, stride, stride)
    out = _conv(h, *c1, (3, 3, 3), s3, (1, 1, 1), relu=True)
    if ds is not None:
        res = _conv(h, *ds, (1, 1, 1), s3, (0, 0, 0), relu=False)
    else:
        res = h
    return _conv(out, *c2, (3, 3, 3), (1, 1, 1), (1, 1, 1),
                 relu=True, residual=res)


def _head_body(x_ref, w0_ref, b0_ref, w1_ref, b1_ref, w2_ref, b2_ref, o_ref):
    e = jnp.dot(x_ref[...], w0_ref[...],
                preferred_element_type=jnp.float32) + b0_ref[...]
    e = jnp.dot(e, w1_ref[...], preferred_element_type=jnp.float32) + b1_ref[...]
    o_ref[...] = jnp.dot(e, w2_ref[...],
                         preferred_element_type=jnp.float32) + b2_ref[...]


def _finetune_head(pool, con0_w, con0_b, con1_w, con1_b, ft_w, ft_b):
    M = pool.shape[0]
    Mp = _rup(M, 8)

    def pw(w):
        return jnp.pad(w.astype(jnp.float32),
                       ((0, 128 - w.shape[0]), (0, 128 - w.shape[1])))

    def pb(v):
        return jnp.pad(v.astype(jnp.float32),
                       (0, 128 - v.shape[0])).reshape(1, 128)

    x_p = jnp.pad(pool.astype(jnp.float32), ((0, Mp - M), (0, 0)))
    args = [x_p, pw(con0_w), pb(con0_b), pw(con1_w), pb(con1_b),
            pw(ft_w), pb(ft_b)]
    out = pl.pallas_call(
        _head_body,
        out_shape=jax.ShapeDtypeStruct((Mp, 128), jnp.float32),
        grid=(1,),
        in_specs=[pl.BlockSpec(a.shape, lambda i: (0, 0)) for a in args],
        out_specs=pl.BlockSpec((Mp, 128), lambda i: (0, 0)),
    )(*args)
    return out[:M, :1]


def kernel(x, stem_w, stem_gamma, stem_beta, l1b0_c1_w, l1b0_c1_g, l1b0_c1_b, l1b0_c2_w, l1b0_c2_g, l1b0_c2_b, l1b1_c1_w, l1b1_c1_g, l1b1_c1_b, l1b1_c2_w, l1b1_c2_g, l1b1_c2_b, l2b0_c1_w, l2b0_c1_g, l2b0_c1_b, l2b0_c2_w, l2b0_c2_g, l2b0_c2_b, l2b0_ds_w, l2b0_ds_g, l2b0_ds_b, l2b1_c1_w, l2b1_c1_g, l2b1_c1_b, l2b1_c2_w, l2b1_c2_g, l2b1_c2_b, l3b0_c1_w, l3b0_c1_g, l3b0_c1_b, l3b0_c2_w, l3b0_c2_g, l3b0_c2_b, l3b0_ds_w, l3b0_ds_g, l3b0_ds_b, l3b1_c1_w, l3b1_c1_g, l3b1_c1_b, l3b1_c2_w, l3b1_c2_g, l3b1_c2_b, l4b0_c1_w, l4b0_c1_g, l4b0_c1_b, l4b0_c2_w, l4b0_c2_g, l4b0_c2_b, l4b0_ds_w, l4b0_ds_g, l4b0_ds_b, l4b1_c1_w, l4b1_c1_g, l4b1_c1_b, l4b1_c2_w, l4b1_c2_g, l4b1_c2_b, fc_w, fc_g, fc_b, cls0_w, cls0_b, cls1_w, cls1_b, cls2_w, cls2_b, con0_w, con0_b, con1_w, con1_b, ft_w, ft_b):
    h = jnp.transpose(x, (0, 2, 3, 4, 1)).astype(jnp.bfloat16)
    h = _conv(h, stem_w, stem_gamma, stem_beta,
              (7, 7, 7), (2, 2, 2), (3, 3, 3), relu=True)

    h = _basic_block(h, (l1b0_c1_w, l1b0_c1_g, l1b0_c1_b),
                     (l1b0_c2_w, l1b0_c2_g, l1b0_c2_b), None, 1)
    h = _basic_block(h, (l1b1_c1_w, l1b1_c1_g, l1b1_c1_b),
                     (l1b1_c2_w, l1b1_c2_g, l1b1_c2_b), None, 1)

    h = _basic_block(h, (l2b0_c1_w, l2b0_c1_g, l2b0_c1_b),
                     (l2b0_c2_w, l2b0_c2_g, l2b0_c2_b),
                     (l2b0_ds_w, l2b0_ds_g, l2b0_ds_b), 2)
    h = _basic_block(h, (l2b1_c1_w, l2b1_c1_g, l2b1_c1_b),
                     (l2b1_c2_w, l2b1_c2_g, l2b1_c2_b), None, 1)

    h = _basic_block(h, (l3b0_c1_w, l3b0_c1_g, l3b0_c1_b),
                     (l3b0_c2_w, l3b0_c2_g, l3b0_c2_b),
                     (l3b0_ds_w, l3b0_ds_g, l3b0_ds_b), 2)
    h = _basic_block(h, (l3b1_c1_w, l3b1_c1_g, l3b1_c1_b),
                     (l3b1_c2_w, l3b1_c2_g, l3b1_c2_b), None, 1)

    h = _basic_block(h, (l4b0_c1_w, l4b0_c1_g, l4b0_c1_b),
                     (l4b0_c2_w, l4b0_c2_g, l4b0_c2_b),
                     (l4b0_ds_w, l4b0_ds_g, l4b0_ds_b), 2)
    h = _basic_block(h, (l4b1_c1_w, l4b1_c1_g, l4b1_c1_b),
                     (l4b1_c2_w, l4b1_c2_g, l4b1_c2_b), None, 1)

    h = _conv(h, fc_w, fc_g, fc_b, (1, 1, 1), (1, 1, 1), (0, 0, 0), relu=True)
    pool = jnp.mean(h.astype(jnp.float32), axis=(1, 2, 3))
    return _finetune_head(pool, con0_w, con0_b, con1_w, con1_b, ft_w, ft_b)
